# baseline (speedup 1.0000x reference)
import functools

import jax
import jax.numpy as jnp
from jax import lax
from jax.experimental import pallas as pl
from jax.experimental.pallas import tpu as pltpu
from jax.experimental.pallas import tpu_sc as plsc

_NUM_USERS = 60000
_NUM_ITEMS = 40000
_N = _NUM_USERS + _NUM_ITEMS
_D = 32
_H = 16
_E = 1600000
_CHUNK = 128
_SUBCORES = 16
_EDGE_PER_TILE = 102400
_E_PAD = _EDGE_PER_TILE * _SUBCORES
_NCHUNKS = _EDGE_PER_TILE // _CHUNK
_N_PAD = 100352
_ROWS_PER_TILE = _N_PAD // _SUBCORES
_ZROWS = 98


_SUBS_PER_TILE = _EDGE_PER_TILE // _CHUNK
_SUP = 8
_NSUP = _SUBS_PER_TILE // _SUP


def _sc_spmm_body(embA, embB, packed, vals, out,
                  acc, ebuf, vbuf, rows, zbuf, t2,
                  lsem, vsem, gsem0, gsem1, gsem2, gsem3,
                  ssem0, ssem1, ssem2, ssem3):
    cid = lax.axis_index("c")
    sid = lax.axis_index("s")

    def _zb(i, _):
        zbuf[i, :] = jnp.zeros((16,), jnp.float32)
        return 0
    lax.fori_loop(0, _ZROWS, _zb, 0)
    r0 = sid * _ROWS_PER_TILE
    for k in range(_ROWS_PER_TILE // _ZROWS):
        pltpu.sync_copy(zbuf, acc.at[pl.ds(r0 + k * _ZROWS, _ZROWS)])
    plsc.subcore_barrier()

    tile_base = sid * _SUBS_PER_TILE

    def _lin_src(sup):
        return packed.at[:, pl.ds(tile_base + sup * _SUP, _SUP), :]

    def _val_src(sup):
        return vals.at[pl.ds(tile_base + sup * _SUP, _SUP), :]

    def _start_lin(sup):
        pltpu.async_copy(_lin_src(sup), ebuf.at[sup & 3], lsem)
        pltpu.async_copy(_val_src(sup), vbuf.at[sup & 3], vsem)

    def _wait_lin(sup):
        pltpu.make_async_copy(_lin_src(0), ebuf.at[sup & 3], lsem).wait()
        pltpu.make_async_copy(_val_src(0), vbuf.at[sup & 3], vsem).wait()

    def _run(emb):
        gsems = (gsem0, gsem1, gsem2, gsem3)
        ssems = (ssem0, ssem1, ssem2, ssem3)

        def _gat(m, rb):
            return pltpu.make_async_copy(
                emb.at[ebuf.at[(m >> 3) & 3, 0, m & 7]],
                rows.at[rb], gsems[rb])

        def _sca(m, rb):
            return pltpu.make_async_copy(
                rows.at[rb],
                acc.at[ebuf.at[(m >> 3) & 3, 1, m & 7]], ssems[rb])

        _start_lin(0)
        _wait_lin(0)
        _start_lin(1)
        for mm in range(3):
            _gat(mm, mm).start()

        def _sub(m, rb, r):
            _gat(m, rb).wait()

            @pl.when(m + 3 < _SUBS_PER_TILE)
            def _():
                def _starts():
                    if r == 0:
                        @pl.when((m & 7) == 4)
                        def _():
                            sup = m >> 3

                            @pl.when(sup < _NSUP - 1)
                            def _():
                                _wait_lin(sup + 1)

                                @pl.when(sup < _NSUP - 2)
                                def _():
                                    _start_lin(sup + 2)
                    _gat(m + 3, (rb + 3) & 3).start()
                if r == 0:
                    @pl.when(m >= 1)
                    def _():
                        _sca(m - 1, (rb + 3) & 3).wait()
                    _starts()
                else:
                    _sca(m - 1, (rb + 3) & 3).wait()
                    _starts()

            b = (m >> 3) & 3

            def _mul(g, _):
                vv = vbuf[b, m & 7, pl.ds(g * 16, 16)]
                for j in range(16):
                    e = g * 16 + j
                    s = lax.broadcast_in_dim(
                        lax.slice_in_dim(vv, j, j + 1), (16,), (0,))
                    rows[rb, e, :] = rows[rb, e, :] * s
                return 0
            lax.fori_loop(0, _CHUNK // 16, _mul, 0)

            _sca(m, rb).start(add=True)

        def _quad(t, _):
            for r in range(4):
                _sub(4 * t + r, r, r % 4)
            return 0
        lax.fori_loop(0, _SUBS_PER_TILE // 4, _quad, 0)

        for mm in range(_SUBS_PER_TILE - 4, _SUBS_PER_TILE):
            _sca(mm, mm & 3).wait()

    @pl.when(cid == 0)
    def _():
        _run(embA)

    @pl.when(cid == 1)
    def _():
        _run(embB)

    plsc.subcore_barrier()

    w0 = sid * (_ROWS_PER_TILE // 8)

    def _wb(it, _):
        pltpu.sync_copy(acc.at[pl.ds(r0 + it * _CHUNK, _CHUNK)], rows.at[0])
        for o in range(_H):
            for j in range(8):
                t2[o, pl.ds(j * _H, _H)] = rows[0, o * 8 + j, :]
        pltpu.sync_copy(t2, out.at[cid, pl.ds(w0 + it * _H, _H)])
        return 0
    lax.fori_loop(0, _ROWS_PER_TILE // _CHUNK, _wb, 0)


def _sc_spmm(embA, embB, packed, vals):
    mesh = plsc.VectorSubcoreMesh(core_axis_name="c", subcore_axis_name="s")
    f = functools.partial(
        pl.kernel,
        mesh=mesh,
        compiler_params=pltpu.CompilerParams(use_tc_tiling_on_sc=False),
        out_type=jax.ShapeDtypeStruct((2, _N_PAD // 8, 128), jnp.float32),
        scratch_types=[
            pltpu.VMEM_SHARED((_N_PAD, _H), jnp.float32),
            pltpu.VMEM((4, 2, _SUP, _CHUNK), jnp.int32),
            pltpu.VMEM((4, _SUP, _CHUNK), jnp.float32),
            pltpu.VMEM((4, _CHUNK, _H), jnp.float32),
            pltpu.VMEM((_ZROWS, _H), jnp.float32),
            pltpu.VMEM((_H, 8 * _H), jnp.float32),
            pltpu.SemaphoreType.DMA,
            pltpu.SemaphoreType.DMA,
            pltpu.SemaphoreType.DMA,
            pltpu.SemaphoreType.DMA,
            pltpu.SemaphoreType.DMA,
            pltpu.SemaphoreType.DMA,
            pltpu.SemaphoreType.DMA,
            pltpu.SemaphoreType.DMA,
            pltpu.SemaphoreType.DMA,
            pltpu.SemaphoreType.DMA,
        ],
    )(_sc_spmm_body)
    return f(embA, embB, packed, vals)


def _fuse_body(sA, sB, eA, eB, W, bias, outA, outB, outN):
    sa = sA[0]
    sb = sB[0]
    ea = eA[...]
    eb = eB[...]
    lhs = jnp.concatenate([sa, sb, sa * ea, sb * eb], axis=1)
    x = jnp.dot(lhs, W[...], preferred_element_type=jnp.float32) + bias[...]
    act = jnp.where(x >= 0, x, 0.2 * x)
    ss = jnp.sum(act * act, axis=1, keepdims=True)
    nm = act * lax.rsqrt(jnp.maximum(ss, 1e-24))
    outA[...] = act[:, 0:16]
    outB[...] = act[:, 16:32]
    outN[...] = nm


_FB = 6272


def _tc_fuse(side, eA, eB, Wc, bc, We, be):
    grid = (_N_PAD // _FB,)
    W = jnp.concatenate([Wc.T, We.T], axis=0)
    bias = (bc + be).reshape(1, _D)
    return pl.pallas_call(
        _fuse_body,
        grid=grid,
        in_specs=[
            pl.BlockSpec((1, _FB, _H), lambda i: (0, i, 0)),
            pl.BlockSpec((1, _FB, _H), lambda i: (1, i, 0)),
            pl.BlockSpec((_FB, _H), lambda i: (i, 0)),
            pl.BlockSpec((_FB, _H), lambda i: (i, 0)),
            pl.BlockSpec((2 * _D, _D), lambda i: (0, 0)),
            pl.BlockSpec((1, _D), lambda i: (0, 0)),
        ],
        out_specs=[
            pl.BlockSpec((_FB, _H), lambda i: (i, 0)),
            pl.BlockSpec((_FB, _H), lambda i: (i, 0)),
            pl.BlockSpec((_FB, _D), lambda i: (i, 0)),
        ],
        out_shape=[
            jax.ShapeDtypeStruct((_N_PAD, _H), jnp.float32),
            jax.ShapeDtypeStruct((_N_PAD, _H), jnp.float32),
            jax.ShapeDtypeStruct((_N_PAD, _D), jnp.float32),
        ],
    )(side, side, eA, eB, W, bias)


def _rating_body(U, T, out):
    u = U[...]
    t = T[...]
    r = lax.dot_general(u, t, dimension_numbers=(((1,), (1,)), ((), ())),
                        preferred_element_type=jnp.float32)
    out[...] = jax.nn.sigmoid(r)


_UB = 64


def _tc_rating(U, T):
    batch, k = U.shape
    grid = (batch // _UB,)
    return pl.pallas_call(
        _rating_body,
        grid=grid,
        in_specs=[
            pl.BlockSpec((_UB, k), lambda i: (i, 0)),
            pl.BlockSpec((_NUM_ITEMS, k), lambda i: (0, 0)),
        ],
        out_specs=pl.BlockSpec((_UB, _NUM_ITEMS), lambda i: (i, 0)),
        out_shape=jax.ShapeDtypeStruct((batch, _NUM_ITEMS), jnp.float32),
    )(U, T)


def kernel(user_emb, item_emb, Wc0, bc0, We0, be0, Wc1, bc1, We1, be1,
           graph_values, graph_indices, users):
    all_emb0 = jnp.concatenate([user_emb, item_emb], axis=0)
    row = graph_indices[0]
    col = graph_indices[1]
    pad = _E_PAD - _E
    pad_idx = (jnp.arange(pad, dtype=jnp.int32) * 64) % _N
    col_p = jnp.concatenate([col, pad_idx])
    row_p = jnp.concatenate([row, pad_idx])
    val_p = jnp.concatenate([graph_values, jnp.zeros((pad,), jnp.float32)])
    packed = jnp.stack([col_p, row_p]).reshape(2, _E_PAD // _CHUNK, _CHUNK)
    vals = val_p.reshape(_E_PAD // _CHUNK, _CHUNK)

    zpad = jnp.zeros((_N_PAD - _N, _D), jnp.float32)
    all_emb_p = jnp.concatenate([all_emb0, zpad], axis=0)
    eA0 = all_emb_p[:, 0:16]
    eB0 = all_emb_p[:, 16:32]

    side0 = _sc_spmm(eA0, eB0, packed, vals).reshape(2, _N_PAD, _H)
    e1A, e1B, norm1 = _tc_fuse(side0, eA0, eB0, Wc0, bc0, We0, be0)
    side1 = _sc_spmm(e1A, e1B, packed, vals).reshape(2, _N_PAD, _H)
    _, _, norm2 = _tc_fuse(side1, e1A, e1B, Wc1, bc1, We1, be1)

    u0 = jnp.take(all_emb0[:_NUM_USERS], users, axis=0)
    u1 = jnp.take(norm1[:_NUM_USERS], users, axis=0)
    u2 = jnp.take(norm2[:_NUM_USERS], users, axis=0)
    U = jnp.concatenate([u0, u1, u2], axis=1)
    T = jnp.concatenate([all_emb0[_NUM_USERS:], norm1[_NUM_USERS:_N],
                         norm2[_NUM_USERS:_N]], axis=1)
    return _tc_rating(U, T)


# speedup vs baseline: 1.0166x; 1.0166x over previous
import functools

import jax
import jax.numpy as jnp
from jax import lax
from jax.experimental import pallas as pl
from jax.experimental.pallas import tpu as pltpu
from jax.experimental.pallas import tpu_sc as plsc

_NUM_USERS = 60000
_NUM_ITEMS = 40000
_N = _NUM_USERS + _NUM_ITEMS
_D = 32
_H = 16
_E = 1600000
_CHUNK = 128
_SUBCORES = 16
_EDGE_PER_TILE = 102400
_E_PAD = _EDGE_PER_TILE * _SUBCORES
_NCHUNKS = _EDGE_PER_TILE // _CHUNK
_N_PAD = 100352
_ROWS_PER_TILE = _N_PAD // _SUBCORES
_ZROWS = 392


_SUBS_PER_TILE = _EDGE_PER_TILE // _CHUNK
_SUP = 8
_NSUP = _SUBS_PER_TILE // _SUP


def _sc_spmm_body(embA, embB, packed, vals, out,
                  acc, ebuf, vbuf, rows, zbuf,
                  lsem, vsem, gsem0, gsem1, gsem2, gsem3,
                  ssem0, ssem1, ssem2, ssem3):
    cid = lax.axis_index("c")
    sid = lax.axis_index("s")

    def _zb(i, _):
        zbuf[i, :] = jnp.zeros((16,), jnp.float32)
        return 0
    lax.fori_loop(0, _ZROWS, _zb, 0)
    r0 = sid * _ROWS_PER_TILE
    zcp = [pltpu.make_async_copy(
        zbuf, acc.at[pl.ds(r0 + k * _ZROWS, _ZROWS)], lsem)
        for k in range(_ROWS_PER_TILE // _ZROWS)]
    for c in zcp:
        c.start()
    for c in zcp:
        c.wait()
    plsc.subcore_barrier()

    tile_base = sid * _SUBS_PER_TILE

    def _lin_src(sup):
        return packed.at[:, pl.ds(tile_base + sup * _SUP, _SUP), :]

    def _val_src(sup):
        return vals.at[pl.ds(tile_base + sup * _SUP, _SUP), :]

    def _start_lin(sup):
        pltpu.async_copy(_lin_src(sup), ebuf.at[sup & 3], lsem)
        pltpu.async_copy(_val_src(sup), vbuf.at[sup & 3], vsem)

    def _wait_lin(sup):
        pltpu.make_async_copy(_lin_src(0), ebuf.at[sup & 3], lsem).wait()
        pltpu.make_async_copy(_val_src(0), vbuf.at[sup & 3], vsem).wait()

    def _run(emb):
        gsems = (gsem0, gsem1, gsem2, gsem3)
        ssems = (ssem0, ssem1, ssem2, ssem3)

        def _gat(m, rb):
            return pltpu.make_async_copy(
                emb.at[ebuf.at[(m >> 3) & 3, 0, m & 7]],
                rows.at[rb], gsems[rb])

        def _sca(m, rb):
            return pltpu.make_async_copy(
                rows.at[rb],
                acc.at[ebuf.at[(m >> 3) & 3, 1, m & 7]], ssems[rb])

        _start_lin(0)
        _wait_lin(0)
        _start_lin(1)
        for mm in range(3):
            _gat(mm, mm).start()

        def _sub(m, rb, r):
            _gat(m, rb).wait()

            @pl.when(m + 3 < _SUBS_PER_TILE)
            def _():
                def _starts():
                    if r == 0:
                        @pl.when((m & 7) == 4)
                        def _():
                            sup = m >> 3

                            @pl.when(sup < _NSUP - 1)
                            def _():
                                _wait_lin(sup + 1)

                                @pl.when(sup < _NSUP - 2)
                                def _():
                                    _start_lin(sup + 2)
                    _gat(m + 3, (rb + 3) & 3).start()
                if r == 0:
                    @pl.when(m >= 1)
                    def _():
                        _sca(m - 1, (rb + 3) & 3).wait()
                    _starts()
                else:
                    _sca(m - 1, (rb + 3) & 3).wait()
                    _starts()

            b = (m >> 3) & 3

            def _mul(g, _):
                vv = vbuf[b, m & 7, pl.ds(g * 16, 16)]
                for j in range(16):
                    e = g * 16 + j
                    s = lax.broadcast_in_dim(
                        lax.slice_in_dim(vv, j, j + 1), (16,), (0,))
                    rows[rb, e, :] = rows[rb, e, :] * s
                return 0
            lax.fori_loop(0, _CHUNK // 16, _mul, 0)

            _sca(m, rb).start(add=True)

        def _quad(t, _):
            for r in range(4):
                _sub(4 * t + r, r, r % 4)
            return 0
        lax.fori_loop(0, _SUBS_PER_TILE // 4, _quad, 0)

        for mm in range(_SUBS_PER_TILE - 4, _SUBS_PER_TILE):
            _sca(mm, mm & 3).wait()

    @pl.when(cid == 0)
    def _():
        _run(embA)

    @pl.when(cid == 1)
    def _():
        _run(embB)

    plsc.subcore_barrier()

    pltpu.sync_copy(acc.at[pl.ds(r0, _ROWS_PER_TILE)],
                    out.at[cid, pl.ds(r0, _ROWS_PER_TILE)])


def _sc_spmm(embA, embB, packed, vals):
    mesh = plsc.VectorSubcoreMesh(core_axis_name="c", subcore_axis_name="s")
    f = functools.partial(
        pl.kernel,
        mesh=mesh,
        compiler_params=pltpu.CompilerParams(use_tc_tiling_on_sc=False),
        out_type=jax.ShapeDtypeStruct((2, _N_PAD, _H), jnp.float32),
        scratch_types=[
            pltpu.VMEM_SHARED((_N_PAD, _H), jnp.float32),
            pltpu.VMEM((4, 2, _SUP, _CHUNK), jnp.int32),
            pltpu.VMEM((4, _SUP, _CHUNK), jnp.float32),
            pltpu.VMEM((4, _CHUNK, _H), jnp.float32),
            pltpu.VMEM((_ZROWS, _H), jnp.float32),
            pltpu.SemaphoreType.DMA,
            pltpu.SemaphoreType.DMA,
            pltpu.SemaphoreType.DMA,
            pltpu.SemaphoreType.DMA,
            pltpu.SemaphoreType.DMA,
            pltpu.SemaphoreType.DMA,
            pltpu.SemaphoreType.DMA,
            pltpu.SemaphoreType.DMA,
            pltpu.SemaphoreType.DMA,
            pltpu.SemaphoreType.DMA,
        ],
    )(_sc_spmm_body)
    return f(embA, embB, packed, vals)


def _fuse_body(sA, sB, eA, eB, W, bias, outA, outB, outN):
    sa = sA[0]
    sb = sB[0]
    ea = eA[...]
    eb = eB[...]
    lhs = jnp.concatenate([sa, sb, sa * ea, sb * eb], axis=1)
    x = jnp.dot(lhs, W[...], preferred_element_type=jnp.float32) + bias[...]
    act = jnp.where(x >= 0, x, 0.2 * x)
    ss = jnp.sum(act * act, axis=1, keepdims=True)
    nm = act * lax.rsqrt(jnp.maximum(ss, 1e-24))
    outA[...] = act[:, 0:16]
    outB[...] = act[:, 16:32]
    outN[...] = nm


_FB = 6272


def _tc_fuse(side, eA, eB, Wc, bc, We, be):
    grid = (_N_PAD // _FB,)
    W = jnp.concatenate([Wc.T, We.T], axis=0)
    bias = (bc + be).reshape(1, _D)
    return pl.pallas_call(
        _fuse_body,
        grid=grid,
        in_specs=[
            pl.BlockSpec((1, _FB, _H), lambda i: (0, i, 0)),
            pl.BlockSpec((1, _FB, _H), lambda i: (1, i, 0)),
            pl.BlockSpec((_FB, _H), lambda i: (i, 0)),
            pl.BlockSpec((_FB, _H), lambda i: (i, 0)),
            pl.BlockSpec((2 * _D, _D), lambda i: (0, 0)),
            pl.BlockSpec((1, _D), lambda i: (0, 0)),
        ],
        out_specs=[
            pl.BlockSpec((_FB, _H), lambda i: (i, 0)),
            pl.BlockSpec((_FB, _H), lambda i: (i, 0)),
            pl.BlockSpec((_FB, _D), lambda i: (i, 0)),
        ],
        out_shape=[
            jax.ShapeDtypeStruct((_N_PAD, _H), jnp.float32),
            jax.ShapeDtypeStruct((_N_PAD, _H), jnp.float32),
            jax.ShapeDtypeStruct((_N_PAD, _D), jnp.float32),
        ],
    )(side, side, eA, eB, W, bias)


def _rating_body(U, T, out):
    u = U[...]
    t = T[...]
    r = lax.dot_general(u, t, dimension_numbers=(((1,), (1,)), ((), ())),
                        preferred_element_type=jnp.float32)
    out[...] = jax.nn.sigmoid(r)


_UB = 64


def _tc_rating(U, T):
    batch, k = U.shape
    grid = (batch // _UB,)
    return pl.pallas_call(
        _rating_body,
        grid=grid,
        in_specs=[
            pl.BlockSpec((_UB, k), lambda i: (i, 0)),
            pl.BlockSpec((_NUM_ITEMS, k), lambda i: (0, 0)),
        ],
        out_specs=pl.BlockSpec((_UB, _NUM_ITEMS), lambda i: (i, 0)),
        out_shape=jax.ShapeDtypeStruct((batch, _NUM_ITEMS), jnp.float32),
    )(U, T)


def kernel(user_emb, item_emb, Wc0, bc0, We0, be0, Wc1, bc1, We1, be1,
           graph_values, graph_indices, users):
    all_emb0 = jnp.concatenate([user_emb, item_emb], axis=0)
    row = graph_indices[0]
    col = graph_indices[1]
    pad = _E_PAD - _E
    pad_idx = (jnp.arange(pad, dtype=jnp.int32) * 64) % _N
    col_p = jnp.concatenate([col, pad_idx])
    row_p = jnp.concatenate([row, pad_idx])
    val_p = jnp.concatenate([graph_values, jnp.zeros((pad,), jnp.float32)])
    packed = jnp.stack([col_p, row_p]).reshape(2, _E_PAD // _CHUNK, _CHUNK)
    vals = val_p.reshape(_E_PAD // _CHUNK, _CHUNK)

    zpad = jnp.zeros((_N_PAD - _N, _D), jnp.float32)
    all_emb_p = jnp.concatenate([all_emb0, zpad], axis=0)
    eA0 = all_emb_p[:, 0:16]
    eB0 = all_emb_p[:, 16:32]

    side0 = _sc_spmm(eA0, eB0, packed, vals)
    e1A, e1B, norm1 = _tc_fuse(side0, eA0, eB0, Wc0, bc0, We0, be0)
    side1 = _sc_spmm(e1A, e1B, packed, vals)
    _, _, norm2 = _tc_fuse(side1, e1A, e1B, Wc1, bc1, We1, be1)

    u0 = jnp.take(all_emb0[:_NUM_USERS], users, axis=0)
    u1 = jnp.take(norm1[:_NUM_USERS], users, axis=0)
    u2 = jnp.take(norm2[:_NUM_USERS], users, axis=0)
    U = jnp.concatenate([u0, u1, u2], axis=1)
    T = jnp.concatenate([all_emb0[_NUM_USERS:], norm1[_NUM_USERS:_N],
                         norm2[_NUM_USERS:_N]], axis=1)
    return _tc_rating(U, T)


# speedup vs baseline: 1.0454x; 1.0283x over previous
import functools

import jax
import jax.numpy as jnp
from jax import lax
from jax.experimental import pallas as pl
from jax.experimental.pallas import tpu as pltpu
from jax.experimental.pallas import tpu_sc as plsc

_NUM_USERS = 60000
_NUM_ITEMS = 40000
_N = _NUM_USERS + _NUM_ITEMS
_D = 32
_H = 16
_E = 1600000
_CHUNK = 128
_SUBCORES = 16
_EDGE_PER_TILE = 102400
_E_PAD = _EDGE_PER_TILE * _SUBCORES
_NCHUNKS = _EDGE_PER_TILE // _CHUNK
_N_PAD = 100352
_ROWS_PER_TILE = _N_PAD // _SUBCORES
_ZROWS = 392


_SUBS_PER_TILE = _EDGE_PER_TILE // _CHUNK
_SUP = 8
_NSUP = _SUBS_PER_TILE // _SUP


def _sc_spmm_body(embA, embB, packed, vals, out,
                  acc, ebuf, vbuf, rows, zbuf,
                  lsem, vsem, gsem0, gsem1, gsem2, gsem3,
                  ssem0, ssem1, ssem2, ssem3):
    cid = lax.axis_index("c")
    sid = lax.axis_index("s")

    def _zb(i, _):
        zbuf[i, :] = jnp.zeros((16,), jnp.float32)
        return 0
    lax.fori_loop(0, _ZROWS, _zb, 0)
    r0 = sid * _ROWS_PER_TILE
    zcp = [pltpu.make_async_copy(
        zbuf, acc.at[pl.ds(r0 + k * _ZROWS, _ZROWS)], lsem)
        for k in range(_ROWS_PER_TILE // _ZROWS)]
    for c in zcp:
        c.start()
    for c in zcp:
        c.wait()
    plsc.subcore_barrier()

    tile_base = sid * _SUBS_PER_TILE

    def _lin_src(sup):
        return packed.at[:, pl.ds(tile_base + sup * _SUP, _SUP), :]

    def _val_src(sup):
        return vals.at[pl.ds(tile_base + sup * _SUP, _SUP), :]

    def _start_lin(sup):
        pltpu.async_copy(_lin_src(sup), ebuf.at[sup & 3], lsem)
        pltpu.async_copy(_val_src(sup), vbuf.at[sup & 3], vsem)

    def _wait_lin(sup):
        pltpu.make_async_copy(_lin_src(0), ebuf.at[sup & 3], lsem).wait()
        pltpu.make_async_copy(_val_src(0), vbuf.at[sup & 3], vsem).wait()

    def _run(emb):
        gsems = (gsem0, gsem1, gsem2, gsem3)
        ssems = (ssem0, ssem1, ssem2, ssem3)

        def _gat(m, rb):
            return pltpu.make_async_copy(
                emb.at[ebuf.at[(m >> 3) & 3, 0, m & 7]],
                rows.at[rb], gsems[rb])

        def _sca(m, rb):
            return pltpu.make_async_copy(
                rows.at[rb],
                acc.at[ebuf.at[(m >> 3) & 3, 1, m & 7]], ssems[rb])

        _start_lin(0)
        _wait_lin(0)
        _start_lin(1)
        for mm in range(3):
            _gat(mm, mm).start()

        def _sub(m, rb, r):
            _gat(m, rb).wait()

            @pl.when(m + 3 < _SUBS_PER_TILE)
            def _():
                def _starts():
                    if r == 0:
                        @pl.when((m & 7) == 4)
                        def _():
                            sup = m >> 3

                            @pl.when(sup < _NSUP - 1)
                            def _():
                                _wait_lin(sup + 1)

                                @pl.when(sup < _NSUP - 2)
                                def _():
                                    _start_lin(sup + 2)
                    _gat(m + 3, (rb + 3) & 3).start()
                if r == 0:
                    @pl.when(m >= 1)
                    def _():
                        _sca(m - 1, (rb + 3) & 3).wait()
                    _starts()
                else:
                    _sca(m - 1, (rb + 3) & 3).wait()
                    _starts()

            b = (m >> 3) & 3

            def _mul(g, _):
                vv = vbuf[b, m & 7, pl.ds(g * 16, 16)]
                for j in range(16):
                    e = g * 16 + j
                    s = lax.broadcast_in_dim(
                        lax.slice_in_dim(vv, j, j + 1), (16,), (0,))
                    rows[rb, e, :] = rows[rb, e, :] * s
                return 0
            lax.fori_loop(0, _CHUNK // 16, _mul, 0)

            _sca(m, rb).start(add=True)

        def _quad(t, _):
            for r in range(4):
                _sub(4 * t + r, r, r % 4)
            return 0
        lax.fori_loop(0, _SUBS_PER_TILE // 4, _quad, 0)

        for mm in range(_SUBS_PER_TILE - 4, _SUBS_PER_TILE):
            _sca(mm, mm & 3).wait()

    @pl.when(cid == 0)
    def _():
        _run(embA)

    @pl.when(cid == 1)
    def _():
        _run(embB)

    plsc.subcore_barrier()

    pltpu.sync_copy(acc.at[pl.ds(r0, _ROWS_PER_TILE)],
                    out.at[cid, pl.ds(r0, _ROWS_PER_TILE)])


def _sc_spmm(embA, embB, packed, vals):
    mesh = plsc.VectorSubcoreMesh(core_axis_name="c", subcore_axis_name="s")
    f = functools.partial(
        pl.kernel,
        mesh=mesh,
        compiler_params=pltpu.CompilerParams(use_tc_tiling_on_sc=False),
        out_type=jax.ShapeDtypeStruct((2, _N_PAD, _H), jnp.float32),
        scratch_types=[
            pltpu.VMEM_SHARED((_N_PAD, _H), jnp.float32),
            pltpu.VMEM((4, 2, _SUP, _CHUNK), jnp.int32),
            pltpu.VMEM((4, _SUP, _CHUNK), jnp.float32),
            pltpu.VMEM((4, _CHUNK, _H), jnp.float32),
            pltpu.VMEM((_ZROWS, _H), jnp.float32),
            pltpu.SemaphoreType.DMA,
            pltpu.SemaphoreType.DMA,
            pltpu.SemaphoreType.DMA,
            pltpu.SemaphoreType.DMA,
            pltpu.SemaphoreType.DMA,
            pltpu.SemaphoreType.DMA,
            pltpu.SemaphoreType.DMA,
            pltpu.SemaphoreType.DMA,
            pltpu.SemaphoreType.DMA,
            pltpu.SemaphoreType.DMA,
        ],
    )(_sc_spmm_body)
    return f(embA, embB, packed, vals)


def _fuse_body(sA, sB, eA, eB, W, bias, outA, outB, outN):
    sa = sA[0]
    sb = sB[0]
    ea = eA[...]
    eb = eB[...]
    lhs = jnp.concatenate([sa, sb, sa * ea, sb * eb], axis=1)
    x = jnp.dot(lhs, W[...], preferred_element_type=jnp.float32) + bias[...]
    act = jnp.where(x >= 0, x, 0.2 * x)
    ss = jnp.sum(act * act, axis=1, keepdims=True)
    nm = act * lax.rsqrt(jnp.maximum(ss, 1e-24))
    outA[...] = act[:, 0:16]
    outB[...] = act[:, 16:32]
    outN[...] = nm


_FB = 5000


def _tc_fuse(side, eA, eB, Wc, bc, We, be):
    grid = (_N // _FB,)
    W = jnp.concatenate([Wc.T, We.T], axis=0)
    bias = (bc + be).reshape(1, _D)
    return pl.pallas_call(
        _fuse_body,
        grid=grid,
        in_specs=[
            pl.BlockSpec((1, _FB, _H), lambda i: (0, i, 0)),
            pl.BlockSpec((1, _FB, _H), lambda i: (1, i, 0)),
            pl.BlockSpec((_FB, _H), lambda i: (i, 0)),
            pl.BlockSpec((_FB, _H), lambda i: (i, 0)),
            pl.BlockSpec((2 * _D, _D), lambda i: (0, 0)),
            pl.BlockSpec((1, _D), lambda i: (0, 0)),
        ],
        out_specs=[
            pl.BlockSpec((_FB, _H), lambda i: (i, 0)),
            pl.BlockSpec((_FB, _H), lambda i: (i, 0)),
            pl.BlockSpec((_FB, _D), lambda i: (i, 0)),
        ],
        out_shape=[
            jax.ShapeDtypeStruct((_N, _H), jnp.float32),
            jax.ShapeDtypeStruct((_N, _H), jnp.float32),
            jax.ShapeDtypeStruct((_N, _D), jnp.float32),
        ],
    )(side, side, eA, eB, W, bias)


def _rating_body(U, T, out):
    u = U[...]
    t = T[...]
    r = lax.dot_general(u, t, dimension_numbers=(((1,), (1,)), ((), ())),
                        preferred_element_type=jnp.float32)
    out[...] = jax.nn.sigmoid(r)


_UB = 64


def _tc_rating(U, T):
    batch, k = U.shape
    grid = (batch // _UB,)
    return pl.pallas_call(
        _rating_body,
        grid=grid,
        in_specs=[
            pl.BlockSpec((_UB, k), lambda i: (i, 0)),
            pl.BlockSpec((_NUM_ITEMS, k), lambda i: (0, 0)),
        ],
        out_specs=pl.BlockSpec((_UB, _NUM_ITEMS), lambda i: (i, 0)),
        out_shape=jax.ShapeDtypeStruct((batch, _NUM_ITEMS), jnp.float32),
    )(U, T)


def kernel(user_emb, item_emb, Wc0, bc0, We0, be0, Wc1, bc1, We1, be1,
           graph_values, graph_indices, users):
    all_emb0 = jnp.concatenate([user_emb, item_emb], axis=0)
    row = graph_indices[0]
    col = graph_indices[1]
    pad = _E_PAD - _E
    pad_idx = (jnp.arange(pad, dtype=jnp.int32) * 64) % _N
    col_p = jnp.concatenate([col, pad_idx])
    row_p = jnp.concatenate([row, pad_idx])
    val_p = jnp.concatenate([graph_values, jnp.zeros((pad,), jnp.float32)])
    packed = jnp.stack([col_p, row_p]).reshape(2, _E_PAD // _CHUNK, _CHUNK)
    vals = val_p.reshape(_E_PAD // _CHUNK, _CHUNK)

    eA0 = all_emb0[:, 0:16]
    eB0 = all_emb0[:, 16:32]

    side0 = _sc_spmm(eA0, eB0, packed, vals)
    e1A, e1B, norm1 = _tc_fuse(side0, eA0, eB0, Wc0, bc0, We0, be0)
    side1 = _sc_spmm(e1A, e1B, packed, vals)
    _, _, norm2 = _tc_fuse(side1, e1A, e1B, Wc1, bc1, We1, be1)

    u0 = jnp.take(all_emb0[:_NUM_USERS], users, axis=0)
    u1 = jnp.take(norm1[:_NUM_USERS], users, axis=0)
    u2 = jnp.take(norm2[:_NUM_USERS], users, axis=0)
    U = jnp.concatenate([u0, u1, u2], axis=1)
    T = jnp.concatenate([all_emb0[_NUM_USERS:], norm1[_NUM_USERS:],
                         norm2[_NUM_USERS:]], axis=1)
    return _tc_rating(U, T)
